# baseline (speedup 1.0000x reference)
import functools

import jax
import jax.numpy as jnp
from jax import lax
from jax.experimental import pallas as pl
from jax.experimental.pallas import tpu as pltpu
from jax.experimental.pallas import tpu_sc as plsc

NC = 2
NS = 16
NW = NC * NS
CH = 128


@functools.cache
def _make_sc_gather(B, DU, DI, VU, VI):
    BPW = B // NW
    NCH = BPW // CH
    mesh = plsc.VectorSubcoreMesh(core_axis_name="c", subcore_axis_name="s")

    @functools.partial(
        pl.kernel,
        mesh=mesh,
        compiler_params=pltpu.CompilerParams(use_tc_tiling_on_sc=False),
        out_type=(
            jax.ShapeDtypeStruct((B // CH, CH, DU), jnp.float32),
            jax.ShapeDtypeStruct((B // CH, CH, DI), jnp.float32),
        ),
        scratch_types=[
            pltpu.VMEM((NCH, CH), jnp.int32),
            pltpu.VMEM((NCH, CH), jnp.int32),
            pltpu.VMEM((NCH, CH, DU), jnp.float32),
            pltpu.VMEM((NCH, CH, DI), jnp.float32),
            pltpu.SemaphoreType.DMA,
        ],
    )
    def sc_gather(uid_hbm, iid_hbm, utab_hbm, itab_hbm, ue_hbm, ie_hbm,
                  uidx_v, iidx_v, urows_v, irows_v, sem):
        wid = lax.axis_index("s") * NC + lax.axis_index("c")
        row0 = wid * NCH
        pltpu.sync_copy(uid_hbm.at[pl.ds(row0, NCH)], uidx_v)
        pltpu.sync_copy(iid_hbm.at[pl.ds(row0, NCH)], iidx_v)
        copies = []
        for j in range(NCH):
            copies.append(
                pltpu.async_copy(utab_hbm.at[uidx_v.at[j]], urows_v.at[j], sem))
            copies.append(
                pltpu.async_copy(itab_hbm.at[iidx_v.at[j]], irows_v.at[j], sem))
        for c in copies:
            c.wait()
        pltpu.sync_copy(urows_v, ue_hbm.at[pl.ds(row0, NCH)])
        pltpu.sync_copy(irows_v, ie_hbm.at[pl.ds(row0, NCH)])

    return sc_gather


def _dense_body(ue_ref, ie_ref, uf_ref, wuf_ref, buf_ref, wt_ref, bt_ref,
                out_ref, *, DU):
    uft = jnp.dot(uf_ref[...], wuf_ref[...],
                  preferred_element_type=jnp.float32) + buf_ref[...]
    wt = wt_ref[...]
    acc = jnp.dot(ue_ref[...], wt[0:DU], preferred_element_type=jnp.float32)
    acc = acc + jnp.dot(uft, wt[DU:2 * DU], preferred_element_type=jnp.float32)
    acc = acc + jnp.dot(ie_ref[...], wt[2 * DU:],
                        preferred_element_type=jnp.float32)
    out_ref[...] = acc + bt_ref[...]


@functools.cache
def _make_tc_dense(B, DU, DI, IU, T, BLK=2048):
    grid = B // BLK
    return pl.pallas_call(
        functools.partial(_dense_body, DU=DU),
        grid=(grid,),
        in_specs=[
            pl.BlockSpec((BLK, DU), lambda i: (i, 0)),
            pl.BlockSpec((BLK, DI), lambda i: (i, 0)),
            pl.BlockSpec((BLK, IU), lambda i: (i, 0)),
            pl.BlockSpec((IU, DU), lambda i: (0, 0)),
            pl.BlockSpec((1, DU), lambda i: (0, 0)),
            pl.BlockSpec((2 * DU + DI, T), lambda i: (0, 0)),
            pl.BlockSpec((1, T), lambda i: (0, 0)),
        ],
        out_specs=pl.BlockSpec((BLK, T), lambda i: (i, 0)),
        out_shape=jax.ShapeDtypeStruct((B, T), jnp.float32),
    )


def kernel(user_id, user_features, item_id, user_table, item_table,
           W_uf, b_uf, W_task, b_task):
    B = user_id.shape[0]
    VU, DU = user_table.shape
    VI, DI = item_table.shape
    IU = user_features.shape[1]
    T = W_task.shape[1]
    uid = user_id.astype(jnp.int32).reshape(B // CH, CH)
    iid = item_id.astype(jnp.int32).reshape(B // CH, CH)
    ue, ie = _make_sc_gather(B, DU, DI, VU, VI)(uid, iid, user_table,
                                                item_table)
    return _make_tc_dense(B, DU, DI, IU, T)(
        ue.reshape(B, DU), ie.reshape(B, DI), user_features, W_uf,
        b_uf.reshape(1, DU), W_task, b_task.reshape(1, T))


# speedup vs baseline: 1.5814x; 1.5814x over previous
import functools

import jax
import jax.numpy as jnp
from jax import lax
from jax.experimental import pallas as pl
from jax.experimental.pallas import tpu as pltpu
from jax.experimental.pallas import tpu_sc as plsc

NC = 2
NS = 16
NW = NC * NS
CH = 128


@functools.cache
def _make_sc_gather(B, DU, DI, VU, VI):
    BPW = B // NW
    mesh = plsc.VectorSubcoreMesh(core_axis_name="c", subcore_axis_name="s")

    @functools.partial(
        pl.kernel,
        mesh=mesh,
        out_type=(
            jax.ShapeDtypeStruct((B, DU), jnp.float32),
            jax.ShapeDtypeStruct((B, DI), jnp.float32),
        ),
        scratch_types=[
            pltpu.VMEM((BPW,), jnp.int32),
            pltpu.VMEM((BPW,), jnp.int32),
            pltpu.VMEM((BPW // 2, DU), jnp.float32),
            pltpu.VMEM((BPW // 2, DI), jnp.float32),
            pltpu.SemaphoreType.DMA,
            pltpu.SemaphoreType.DMA,
        ],
    )
    def sc_gather(uid_hbm, iid_hbm, utab_hbm, itab_hbm, ue_hbm, ie_hbm,
                  uidx_v, iidx_v, urows_v, irows_v, sem_u, sem_i):
        wid = lax.axis_index("s") * NC + lax.axis_index("c")
        base = wid * BPW
        HB = BPW // 2
        pltpu.sync_copy(uid_hbm.at[pl.ds(base, BPW)], uidx_v)
        pltpu.sync_copy(iid_hbm.at[pl.ds(base, BPW)], iidx_v)

        for h in range(2):
            def body_u(c, carry):
                s0 = h * HB + c * 16
                d0 = c * 16
                uvec = uidx_v[pl.ds(s0, 16)]
                for j in range(16):
                    pltpu.async_copy(utab_hbm.at[pl.ds(uvec[j], 1)],
                                     urows_v.at[pl.ds(d0 + j, 1)], sem_u)
                return carry

            def body_i(c, carry):
                s0 = h * HB + c * 16
                d0 = c * 16
                ivec = iidx_v[pl.ds(s0, 16)]
                for j in range(16):
                    pltpu.async_copy(itab_hbm.at[pl.ds(ivec[j], 1)],
                                     irows_v.at[pl.ds(d0 + j, 1)], sem_i)
                return carry

            lax.fori_loop(0, HB // 16, body_u, 0)
            lax.fori_loop(0, HB // 16, body_i, 0)
            pltpu.make_async_copy(utab_hbm.at[pl.ds(0, HB)], urows_v,
                                  sem_u).wait()
            pltpu.make_async_copy(itab_hbm.at[pl.ds(0, HB)], irows_v,
                                  sem_i).wait()

            WB = 64

            def wb(k, carry):
                r0 = pl.multiple_of(k * WB, WB)
                pltpu.sync_copy(urows_v.at[pl.ds(r0, WB)],
                                ue_hbm.at[pl.ds(base + h * HB + r0, WB)])
                pltpu.sync_copy(irows_v.at[pl.ds(r0, WB)],
                                ie_hbm.at[pl.ds(base + h * HB + r0, WB)])
                return carry

            lax.fori_loop(0, HB // WB, wb, 0)

    return sc_gather


def _dense_body(ue_ref, ie_ref, uf_ref, wuf_ref, buf_ref, wt_ref, bt_ref,
                out_ref, *, DU):
    uft = jnp.dot(uf_ref[...], wuf_ref[...],
                  preferred_element_type=jnp.float32) + buf_ref[...]
    wt = wt_ref[...]
    acc = jnp.dot(ue_ref[...], wt[0:DU], preferred_element_type=jnp.float32)
    acc = acc + jnp.dot(uft, wt[DU:2 * DU], preferred_element_type=jnp.float32)
    acc = acc + jnp.dot(ie_ref[...], wt[2 * DU:],
                        preferred_element_type=jnp.float32)
    out_ref[...] = acc + bt_ref[...]


@functools.cache
def _make_tc_dense(B, DU, DI, IU, T, BLK=2048):
    grid = B // BLK
    return pl.pallas_call(
        functools.partial(_dense_body, DU=DU),
        grid=(grid,),
        in_specs=[
            pl.BlockSpec((BLK, DU), lambda i: (i, 0)),
            pl.BlockSpec((BLK, DI), lambda i: (i, 0)),
            pl.BlockSpec((BLK, IU), lambda i: (i, 0)),
            pl.BlockSpec((IU, DU), lambda i: (0, 0)),
            pl.BlockSpec((1, DU), lambda i: (0, 0)),
            pl.BlockSpec((2 * DU + DI, T), lambda i: (0, 0)),
            pl.BlockSpec((1, T), lambda i: (0, 0)),
        ],
        out_specs=pl.BlockSpec((BLK, T), lambda i: (i, 0)),
        out_shape=jax.ShapeDtypeStruct((B, T), jnp.float32),
    )


def kernel(user_id, user_features, item_id, user_table, item_table,
           W_uf, b_uf, W_task, b_task):
    B = user_id.shape[0]
    VU, DU = user_table.shape
    VI, DI = item_table.shape
    IU = user_features.shape[1]
    T = W_task.shape[1]
    uid = user_id.astype(jnp.int32)
    iid = item_id.astype(jnp.int32)
    ue, ie = _make_sc_gather(B, DU, DI, VU, VI)(uid, iid, user_table,
                                                item_table)
    return _make_tc_dense(B, DU, DI, IU, T)(
        ue, ie, user_features, W_uf,
        b_uf.reshape(1, DU), W_task, b_task.reshape(1, T))
